# Initial kernel scaffold; baseline (speedup 1.0000x reference)
import jax
import jax.numpy as jnp
from jax.experimental import pallas as pl


def kernel(wt1, b1, wt2, b2, wt3, b3, wl1, bl1, wl2, bl2, state):
    raise NotImplementedError("write your pallas kernel here")


# speedup vs baseline: 1.1611x; 1.1611x over previous
import jax
import jax.numpy as jnp
from jax.experimental import pallas as pl
from jax.experimental.pallas import tpu as pltpu

_HO1, _S1 = 36, 48
_HO2, _S2 = 18, 32
_HO3, _S3 = 9, 16
_C = 32
_R1 = (_HO1 + 2) * _S1
_R2 = (_HO2 + 2) * _S2
_R3 = (_HO3 + 2) * _S3
_NB = 8


def _taps_gemm(x, wt_ref, b_ref, m, s):
    acc = jnp.dot(x[0:m], wt_ref[0], preferred_element_type=jnp.float32)
    for t, off in ((1, 1), (2, s), (3, s + 1)):
        acc = acc + jnp.dot(x[off:off + m], wt_ref[t],
                            preferred_element_type=jnp.float32)
    return jnp.maximum(acc + b_ref[...], 0.0)


def _resplit_s2d(src_ref, dst_ref, ho, s, rpe):
    dst_ref[...] = jnp.zeros(dst_ref.shape, dst_ref.dtype)
    a = ho // 2
    for pj in (0, 1):
        e = src_ref[pl.ds(pj, _NB * rpe // 2, stride=2), :]
        v = e.reshape(_NB, rpe // 2, _C)[:, :ho * s // 2, :]
        v = v.reshape(_NB, a, 2, s // 2, _C)
        for pi in (0, 1):
            ph, pw = 1 - pi, 1 - pj
            cb = ph * 2 + pw
            dst_ref[:, 1 - ph:1 - ph + a, 1 - pw:1 - pw + a,
                    cb * _C:(cb + 1) * _C] = v[:, :, pi, :a, :]


def _fused_kernel(x_ref, wt1_ref, b1_ref, wt2_ref, b2_ref, wt3_ref, b3_ref,
                  wl1_ref, bl1_ref, wl2_ref, bl2_ref, o_ref,
                  o1_ref, x2_ref, o2_ref, x3_ref, o3_ref):
    m1 = _NB * _R1 - 2 * _S1
    o1_ref[0:m1, :] = _taps_gemm(x_ref[...], wt1_ref, b1_ref, m1, _S1)
    _resplit_s2d(o1_ref, x2_ref, _HO1, _S1, _R1)

    m2 = _NB * _R2 - 2 * _S2
    x2f = x2_ref[...].reshape(_NB * _R2, 4 * _C)
    o2_ref[0:m2, :] = _taps_gemm(x2f, wt2_ref, b2_ref, m2, _S2)
    _resplit_s2d(o2_ref, x3_ref, _HO2, _S2, _R2)

    m3 = _NB * _R3 - 2 * _S3
    x3f = x3_ref[...].reshape(_NB * _R3, 4 * _C)
    o3_ref[0:m3, :] = _taps_gemm(x3f, wt3_ref, b3_ref, m3, _S3)

    vs = []
    for pj in (0, 1):
        e = o3_ref[pl.ds(pj, _NB * _R3 // 2, stride=2), :]
        v = e.reshape(_NB, _R3 // 2, _C)[:, :(_HO3 + 1) * _S3 // 2, :]
        vs.append(v.reshape(_NB, (_HO3 + 1) // 2, 2, _S3 // 2, _C))
    pooled = None
    for kh in range(3):
        for kw in range(3):
            t = vs[kw % 2][:, kh // 2:kh // 2 + 4, kh % 2,
                           kw // 2:kw // 2 + 4, :]
            pooled = t if pooled is None else jnp.maximum(pooled, t)

    feat = pooled.reshape(_NB, 4 * 4 * _C)
    h = jnp.dot(feat, wl1_ref[...], preferred_element_type=jnp.float32)
    h = jnp.maximum(h + bl1_ref[...], 0.0)
    o_ref[...] = (jnp.dot(h, wl2_ref[...], preferred_element_type=jnp.float32)
                  + bl2_ref[...])


def _s2d_prep(x_nchw):
    b, c, h, w = x_nchw.shape
    x = jnp.pad(jnp.transpose(x_nchw, (0, 2, 3, 1)),
                ((0, 0), (1, 1), (1, 1), (0, 0)))
    q = (h + 2) // 2
    x = x.reshape(b, q, 2, q, 2, c)
    x = jnp.transpose(x, (0, 1, 3, 2, 4, 5)).reshape(b, q, q, 4 * c)
    x = jnp.pad(x, ((0, 0), (0, _HO1 + 2 - q), (0, _S1 - q), (0, 0)))
    return x.reshape(b * _R1, 4 * c)


def kernel(wt1, b1, wt2, b2, wt3, b3, wl1, bl1, wl2, bl2, state):
    bsz = state.shape[0]
    c4 = 4 * state.shape[1]
    out_dim = wl2.shape[1]
    x = _s2d_prep(state)
    return pl.pallas_call(
        _fused_kernel,
        out_shape=jax.ShapeDtypeStruct((bsz, out_dim), jnp.float32),
        grid=(bsz // _NB,),
        in_specs=[
            pl.BlockSpec((_NB * _R1, c4), lambda i: (i, 0)),
            pl.BlockSpec(wt1.shape, lambda i: (0, 0, 0)),
            pl.BlockSpec((1, _C), lambda i: (0, 0)),
            pl.BlockSpec(wt2.shape, lambda i: (0, 0, 0)),
            pl.BlockSpec((1, _C), lambda i: (0, 0)),
            pl.BlockSpec(wt3.shape, lambda i: (0, 0, 0)),
            pl.BlockSpec((1, _C), lambda i: (0, 0)),
            pl.BlockSpec(wl1.shape, lambda i: (0, 0)),
            pl.BlockSpec((1, wl1.shape[1]), lambda i: (0, 0)),
            pl.BlockSpec(wl2.shape, lambda i: (0, 0)),
            pl.BlockSpec((1, out_dim), lambda i: (0, 0)),
        ],
        out_specs=pl.BlockSpec((_NB, out_dim), lambda i: (i, 0)),
        scratch_shapes=[
            pltpu.VMEM((_NB * _R1, _C), jnp.float32),
            pltpu.VMEM((_NB, _HO2 + 2, _S2, 4 * _C), jnp.float32),
            pltpu.VMEM((_NB * _R2, _C), jnp.float32),
            pltpu.VMEM((_NB, _HO3 + 2, _S3, 4 * _C), jnp.float32),
            pltpu.VMEM((_NB * _R3, _C), jnp.float32),
        ],
        compiler_params=pltpu.CompilerParams(
            dimension_semantics=("parallel",)),
    )(x, wt1, b1, wt2, b2, wt3, b3, wl1, bl1, wl2, bl2)


# speedup vs baseline: 1.3125x; 1.1304x over previous
import jax
import jax.numpy as jnp
from jax.experimental import pallas as pl
from jax.experimental.pallas import tpu as pltpu

_HO1, _S1 = 36, 40
_HO2, _S2 = 18, 24
_HO3, _S3 = 9, 16
_C = 32
_R1 = (_HO1 + 2) * _S1
_R2 = (_HO2 + 2) * _S2
_R3 = (_HO3 + 2) * _S3
_NB = 8


def _taps_gemm(x, wt_ref, b_ref, m, s):
    acc = jnp.dot(x[0:m], wt_ref[0], preferred_element_type=jnp.float32)
    for t, off in ((1, 1), (2, s), (3, s + 1)):
        acc = acc + jnp.dot(x[off:off + m], wt_ref[t],
                            preferred_element_type=jnp.float32)
    return jnp.maximum(acc + b_ref[...], 0.0)


def _resplit_s2d(src_ref, dst_ref, ho, s, rpe):
    dst_ref[...] = jnp.zeros(dst_ref.shape, dst_ref.dtype)
    a = ho // 2
    for pj in (0, 1):
        e = src_ref[pl.ds(pj, _NB * rpe // 2, stride=2), :]
        v = e.reshape(_NB, rpe // 2, _C)[:, :ho * s // 2, :]
        v = v.reshape(_NB, a, 2, s // 2, _C)
        for pi in (0, 1):
            ph, pw = 1 - pi, 1 - pj
            cb = ph * 2 + pw
            dst_ref[:, 1 - ph:1 - ph + a, 1 - pw:1 - pw + a,
                    cb * _C:(cb + 1) * _C] = v[:, :, pi, :a, :]


def _fused_kernel(x_ref, wt1_ref, b1_ref, wt2_ref, b2_ref, wt3_ref, b3_ref,
                  wl1_ref, bl1_ref, wl2_ref, bl2_ref, o_ref,
                  o1_ref, x2_ref, o2_ref, x3_ref, o3_ref):
    m1 = _NB * _R1 - 2 * _S1
    o1_ref[0:m1, :] = _taps_gemm(x_ref[...], wt1_ref, b1_ref, m1, _S1)
    _resplit_s2d(o1_ref, x2_ref, _HO1, _S1, _R1)

    m2 = _NB * _R2 - 2 * _S2
    x2f = x2_ref[...].reshape(_NB * _R2, 4 * _C)
    o2_ref[0:m2, :] = _taps_gemm(x2f, wt2_ref, b2_ref, m2, _S2)
    _resplit_s2d(o2_ref, x3_ref, _HO2, _S2, _R2)

    m3 = _NB * _R3 - 2 * _S3
    x3f = x3_ref[...].reshape(_NB * _R3, 4 * _C)
    o3_ref[0:m3, :] = _taps_gemm(x3f, wt3_ref, b3_ref, m3, _S3)

    vs = []
    for pj in (0, 1):
        e = o3_ref[pl.ds(pj, _NB * _R3 // 2, stride=2), :]
        v = e.reshape(_NB, _R3 // 2, _C)[:, :(_HO3 + 1) * _S3 // 2, :]
        vs.append(v.reshape(_NB, (_HO3 + 1) // 2, 2, _S3 // 2, _C))
    pooled = None
    for kh in range(3):
        for kw in range(3):
            t = vs[kw % 2][:, kh // 2:kh // 2 + 4, kh % 2,
                           kw // 2:kw // 2 + 4, :]
            pooled = t if pooled is None else jnp.maximum(pooled, t)

    feat = pooled.reshape(_NB, 4 * 4 * _C)
    h = jnp.dot(feat, wl1_ref[...], preferred_element_type=jnp.float32)
    h = jnp.maximum(h + bl1_ref[...], 0.0)
    o_ref[...] = (jnp.dot(h, wl2_ref[...], preferred_element_type=jnp.float32)
                  + bl2_ref[...])


def _s2d_prep(x_nchw):
    b, c, h, w = x_nchw.shape
    x = jnp.pad(jnp.transpose(x_nchw, (0, 2, 3, 1)),
                ((0, 0), (1, 1), (1, 1), (0, 0)))
    q = (h + 2) // 2
    x = x.reshape(b, q, 2, q, 2, c)
    x = jnp.transpose(x, (0, 1, 3, 2, 4, 5)).reshape(b, q, q, 4 * c)
    x = jnp.pad(x, ((0, 0), (0, _HO1 + 2 - q), (0, _S1 - q), (0, 0)))
    return x.reshape(b * _R1, 4 * c)


def kernel(wt1, b1, wt2, b2, wt3, b3, wl1, bl1, wl2, bl2, state):
    bsz = state.shape[0]
    c4 = 4 * state.shape[1]
    out_dim = wl2.shape[1]
    x = _s2d_prep(state)
    return pl.pallas_call(
        _fused_kernel,
        out_shape=jax.ShapeDtypeStruct((bsz, out_dim), jnp.float32),
        grid=(bsz // _NB,),
        in_specs=[
            pl.BlockSpec((_NB * _R1, c4), lambda i: (i, 0)),
            pl.BlockSpec(wt1.shape, lambda i: (0, 0, 0)),
            pl.BlockSpec((1, _C), lambda i: (0, 0)),
            pl.BlockSpec(wt2.shape, lambda i: (0, 0, 0)),
            pl.BlockSpec((1, _C), lambda i: (0, 0)),
            pl.BlockSpec(wt3.shape, lambda i: (0, 0, 0)),
            pl.BlockSpec((1, _C), lambda i: (0, 0)),
            pl.BlockSpec(wl1.shape, lambda i: (0, 0)),
            pl.BlockSpec((1, wl1.shape[1]), lambda i: (0, 0)),
            pl.BlockSpec(wl2.shape, lambda i: (0, 0)),
            pl.BlockSpec((1, out_dim), lambda i: (0, 0)),
        ],
        out_specs=pl.BlockSpec((_NB, out_dim), lambda i: (i, 0)),
        scratch_shapes=[
            pltpu.VMEM((_NB * _R1, _C), jnp.float32),
            pltpu.VMEM((_NB, _HO2 + 2, _S2, 4 * _C), jnp.float32),
            pltpu.VMEM((_NB * _R2, _C), jnp.float32),
            pltpu.VMEM((_NB, _HO3 + 2, _S3, 4 * _C), jnp.float32),
            pltpu.VMEM((_NB * _R3, _C), jnp.float32),
        ],
        compiler_params=pltpu.CompilerParams(
            dimension_semantics=("parallel",)),
    )(x, wt1, b1, wt2, b2, wt3, b3, wl1, bl1, wl2, bl2)
